# baseline (speedup 1.0000x reference)
import functools

import jax
import jax.numpy as jnp
from jax import lax
from jax.experimental import pallas as pl
from jax.experimental.pallas import tpu as pltpu
from jax.experimental.pallas import tpu_sc as plsc

_LANES = 16
_NCORES = 2
_NSUB = 16
_NWORK = _NCORES * _NSUB
_NBUF = 4
_DEPTH = 3
_IW = 128


def _emb_kernel(nf, batch, dim):
    rows = batch * nf
    rows_pw = rows // _NWORK
    nchunks = rows_pw // _IW
    assert rows_pw % _IW == 0

    mesh = plsc.VectorSubcoreMesh(
        core_axis_name="c", subcore_axis_name="s",
        num_cores=_NCORES, num_subcores=_NSUB)

    @functools.partial(
        pl.kernel,
        out_type=jax.ShapeDtypeStruct((rows, dim), jnp.float32),
        mesh=mesh,
        compiler_params=pltpu.CompilerParams(use_tc_tiling_on_sc=False),
        scratch_types=[
            pltpu.VMEM((nchunks, _IW), jnp.int32),
            pltpu.VMEM((_NBUF, _IW, dim), jnp.float32),
        ] + [pltpu.SemaphoreType.DMA] * (2 * _NBUF),
    )
    def emb(idx_hbm, table_hbm, out_hbm, idx_v, rowbuf, *sems):
        gsems, ssems = sems[:_NBUF], sems[_NBUF:]
        wid = lax.axis_index("s") * _NCORES + lax.axis_index("c")
        base = wid * rows_pw

        pltpu.sync_copy(idx_hbm.at[wid], idx_v)

        def gather(j, k):
            return pltpu.make_async_copy(
                table_hbm.at[idx_v.at[j]], rowbuf.at[k], gsems[k])

        def scatter(j, k):
            dst = out_hbm.at[pl.ds(base + j * _IW, _IW)]
            return pltpu.make_async_copy(rowbuf.at[k], dst, ssems[k])

        for j in range(nchunks + _DEPTH):
            if j < nchunks:
                k = j % _NBUF
                if j >= _NBUF:
                    scatter(j - _NBUF, k).wait()
                gather(j, k).start()
            if j >= _DEPTH:
                jj = j - _DEPTH
                kk = jj % _NBUF
                gather(jj, kk).wait()
                scatter(jj, kk).start()
        for jj in range(max(0, nchunks - _NBUF), nchunks):
            scatter(jj, jj % _NBUF).wait()

    return emb


def kernel(values, table, offsets):
    nf = offsets.shape[1]
    batch = values.shape[0] // nf
    dim = table.shape[1]
    idx = (values.reshape(nf, batch).T + offsets).reshape(-1)
    idx3d = idx.reshape(_NWORK, batch * nf // (_NWORK * _IW), _IW)
    emb = _emb_kernel(nf, batch, dim)
    out = emb(idx3d, table)
    return out.reshape(batch, nf, dim)


# speedup vs baseline: 1.0164x; 1.0164x over previous
import functools

import jax
import jax.numpy as jnp
from jax import lax
from jax.experimental import pallas as pl
from jax.experimental.pallas import tpu as pltpu
from jax.experimental.pallas import tpu_sc as plsc

_LANES = 16
_NCORES = 2
_NSUB = 16
_NWORK = _NCORES * _NSUB
_NBUF = 4
_DEPTH = 3


def _emb_kernel(nf, batch, dim):
    bpw = batch // _NWORK
    nvec = bpw // _LANES
    assert batch % _NWORK == 0 and bpw % _LANES == 0

    mesh = plsc.VectorSubcoreMesh(
        core_axis_name="c", subcore_axis_name="s",
        num_cores=_NCORES, num_subcores=_NSUB)

    @functools.partial(
        pl.kernel,
        out_type=jax.ShapeDtypeStruct((batch, nf * dim), jnp.float32),
        mesh=mesh,
        compiler_params=pltpu.CompilerParams(use_tc_tiling_on_sc=False),
        scratch_types=[
            pltpu.VMEM((nf, bpw), jnp.int32),
            pltpu.VMEM((((nf + _LANES - 1) // _LANES) * _LANES,),
                       jnp.int32),
            pltpu.VMEM((_NBUF, bpw, dim), jnp.float32),
        ] + [pltpu.SemaphoreType.DMA] * (2 * _NBUF),
    )
    def emb(values_hbm, table_hbm, offsets_hbm, out_hbm,
            idx_v, offs_s, rowbuf, *sems):
        gsems, ssems = sems[:_NBUF], sems[_NBUF:]
        wid = lax.axis_index("s") * _NCORES + lax.axis_index("c")
        b0 = wid * bpw

        pltpu.sync_copy(values_hbm.at[:, pl.ds(b0, bpw)], idx_v)
        pltpu.sync_copy(offsets_hbm, offs_s.at[pl.ds(0, nf)])

        offgrp = [offs_s[pl.ds(g * _LANES, _LANES)]
                  for g in range((nf + _LANES - 1) // _LANES)]
        for f in range(nf):
            e = f % _LANES
            offv = jnp.broadcast_to(offgrp[f // _LANES][e:e + 1], (_LANES,))
            for i in range(nvec):
                sl = pl.ds(i * _LANES, _LANES)
                idx_v[f, sl] = idx_v[f, sl] + offv

        def gather(f, k):
            return pltpu.make_async_copy(
                table_hbm.at[idx_v.at[f]], rowbuf.at[k], gsems[k])

        def scatter(f, k):
            dst = out_hbm.at[pl.ds(b0, bpw), pl.ds(f * dim, dim)]
            return pltpu.make_async_copy(rowbuf.at[k], dst, ssems[k])

        for j in range(nf + _DEPTH):
            if j < nf:
                k = j % _NBUF
                if j >= _NBUF:
                    scatter(j - _NBUF, k).wait()
                gather(j, k).start()
            if j >= _DEPTH:
                jj = j - _DEPTH
                kk = jj % _NBUF
                gather(jj, kk).wait()
                scatter(jj, kk).start()
        for jj in range(max(0, nf - _NBUF), nf):
            scatter(jj, jj % _NBUF).wait()

    return emb


def kernel(values, table, offsets):
    nf = offsets.shape[1]
    batch = values.shape[0] // nf
    dim = table.shape[1]
    values2d = values.reshape(nf, batch)
    emb = _emb_kernel(nf, batch, dim)
    out = emb(values2d, table, offsets.astype(jnp.int32).reshape(nf))
    return out.reshape(batch, nf, dim)
